# Initial kernel scaffold; baseline (speedup 1.0000x reference)
import jax
import jax.numpy as jnp
from jax.experimental import pallas as pl


def kernel(adj_index, adj_vals, id_adj_index, id_adj_vals, feature_adj_index, feature_adj_vals, image_embedding, text_embedding, image_id, text_id, uEmbeds, iEmbeds, image_trans, text_trans, image_id_trans, text_id_trans, modal_weight):
    raise NotImplementedError("write your pallas kernel here")


# speedup vs baseline: 3.2483x; 3.2483x over previous
import dataclasses
import functools

import jax
import jax.numpy as jnp
from jax import lax
from jax.experimental import pallas as pl
from jax.experimental.pallas import tpu as pltpu
from jax.experimental.pallas import tpu_sc as plsc

_USER = 25000
_ITEM = 25000
_LAT = 64
_HALF = 32
_NC = 2
_NS = 16
_NW = _NC * _NS
_CHUNK = 128
_WIN = 8
_LANES = 16


@functools.lru_cache(maxsize=None)
def _make_sc_spmm(n_src, n_dst, e_pad):
    chunks_ps = e_pad // (_NS * _CHUNK)
    rows_pt = n_dst // _NS
    mesh = plsc.VectorSubcoreMesh(core_axis_name="c", subcore_axis_name="s")

    nwin = chunks_ps // _WIN

    def body(x_hbm, src_hbm, dst_hbm, val_hbm, zero_hbm, out_hbm,
             src_v, dst_v, val_v, rows_v, acc_sh, sem):
        c = lax.axis_index("c")
        s = lax.axis_index("s")
        cbase = s * chunks_ps

        pltpu.sync_copy(zero_hbm.at[pl.ds(s * rows_pt, rows_pt)],
                        acc_sh.at[pl.ds(s * rows_pt, rows_pt)])
        plsc.subcore_barrier()

        @pl.loop(0, nwin)
        def _(w):
            wbase = cbase + w * _WIN
            pltpu.sync_copy(src_hbm.at[pl.ds(wbase, _WIN)], src_v)
            pltpu.sync_copy(dst_hbm.at[pl.ds(wbase, _WIN)], dst_v)
            pltpu.sync_copy(val_hbm.at[pl.ds(wbase, _WIN)], val_v)

            @pl.loop(0, _WIN)
            def _(j):
                pltpu.async_copy(
                    x_hbm.at[c].at[src_v.at[j]], rows_v, sem).wait()

                @pl.loop(0, _CHUNK)
                def _(e):
                    v = plsc.load_gather(
                        val_v, [jnp.full((_LANES,), j, jnp.int32),
                                jnp.full((_LANES,), e, jnp.int32)])
                    rows_v[e, pl.ds(0, _LANES)] = (
                        rows_v[e, pl.ds(0, _LANES)] * v)
                    rows_v[e, pl.ds(_LANES, _LANES)] = (
                        rows_v[e, pl.ds(_LANES, _LANES)] * v)

                pltpu.sync_copy(rows_v, acc_sh.at[dst_v.at[j]], add=True)

        plsc.subcore_barrier()
        pltpu.sync_copy(acc_sh.at[pl.ds(s * rows_pt, rows_pt)],
                        out_hbm.at[c].at[pl.ds(s * rows_pt, rows_pt)])

    cp = pltpu.CompilerParams()
    for field, val in (("needs_layout_passes", False),
                       ("use_tc_tiling_on_sc", False)):
        if field in pltpu.CompilerParams.__dataclass_fields__:
            cp = dataclasses.replace(cp, **{field: val})
    return pl.kernel(
        body,
        out_type=jax.ShapeDtypeStruct((_NC, n_dst, _HALF), jnp.float32),
        mesh=mesh,
        compiler_params=cp,
        scratch_types=[
            pltpu.VMEM((_WIN, _CHUNK), jnp.int32),
            pltpu.VMEM((_WIN, _CHUNK), jnp.int32),
            pltpu.VMEM((_WIN, _CHUNK), jnp.float32),
            pltpu.VMEM((_CHUNK, _HALF), jnp.float32),
            pltpu.VMEM_SHARED((n_dst, _HALF), jnp.float32),
            pltpu.SemaphoreType.DMA,
        ],
    )


def _pad_edges(idx, vals, n_dst):
    e = idx.shape[1]
    group = _NS * _CHUNK * 8
    e_pad = -(-e // group) * group
    pad = e_pad - e
    dst = jnp.concatenate(
        [idx[0].astype(jnp.int32),
         jnp.arange(pad, dtype=jnp.int32) % n_dst])
    src = jnp.concatenate(
        [idx[1].astype(jnp.int32), jnp.zeros((pad,), jnp.int32)])
    v = jnp.concatenate([vals, jnp.zeros((pad,), jnp.float32)])
    return (src.reshape(-1, _CHUNK), dst.reshape(-1, _CHUNK),
            v.reshape(-1, _CHUNK), e_pad)


_FEAT_BLK = 200


def _feats_body(w_ref, img_ref, txt_ref, iid_ref, tid_ref,
                wi_ref, wt_ref, wii_ref, wti_ref, feats_ref, ids_ref):
    w0 = w_ref[0]
    w1 = w_ref[1]

    def leaky(y):
        return jnp.where(y > 0, y, 0.2 * y)

    fi = leaky(jnp.dot(img_ref[...].astype(jnp.bfloat16),
                       wi_ref[...].astype(jnp.bfloat16),
                       preferred_element_type=jnp.float32))
    ft = leaky(jnp.dot(txt_ref[...].astype(jnp.bfloat16),
                       wt_ref[...].astype(jnp.bfloat16),
                       preferred_element_type=jnp.float32))
    af = w0 * fi + w1 * ft
    feats_ref[0] = af[:, :_HALF]
    feats_ref[1] = af[:, _HALF:]

    ii = leaky(jnp.dot(iid_ref[...].astype(jnp.bfloat16),
                       wii_ref[...].astype(jnp.bfloat16),
                       preferred_element_type=jnp.float32))
    ti = leaky(jnp.dot(tid_ref[...].astype(jnp.bfloat16),
                       wti_ref[...].astype(jnp.bfloat16),
                       preferred_element_type=jnp.float32))
    ai = w0 * ii + w1 * ti
    ids_ref[0] = ai[:, :_HALF]
    ids_ref[1] = ai[:, _HALF:]


def _modal_transforms(weight, img, txt, iid, tid, wi, wt, wii, wti):
    nblk = _ITEM // _FEAT_BLK
    return pl.pallas_call(
        _feats_body,
        grid=(nblk,),
        in_specs=[
            pl.BlockSpec(memory_space=pltpu.SMEM),
            pl.BlockSpec((_FEAT_BLK, img.shape[1]), lambda i: (i, 0)),
            pl.BlockSpec((_FEAT_BLK, txt.shape[1]), lambda i: (i, 0)),
            pl.BlockSpec((_FEAT_BLK, _LAT), lambda i: (i, 0)),
            pl.BlockSpec((_FEAT_BLK, _LAT), lambda i: (i, 0)),
            pl.BlockSpec((img.shape[1], _LAT), lambda i: (0, 0)),
            pl.BlockSpec((txt.shape[1], _LAT), lambda i: (0, 0)),
            pl.BlockSpec((_LAT, _LAT), lambda i: (0, 0)),
            pl.BlockSpec((_LAT, _LAT), lambda i: (0, 0)),
        ],
        out_specs=[
            pl.BlockSpec((_NC, _FEAT_BLK, _HALF), lambda i: (0, i, 0)),
            pl.BlockSpec((_NC, _FEAT_BLK, _HALF), lambda i: (0, i, 0)),
        ],
        out_shape=[
            jax.ShapeDtypeStruct((_NC, _ITEM, _HALF), jnp.float32),
            jax.ShapeDtypeStruct((_NC, _ITEM, _HALF), jnp.float32),
        ],
        compiler_params=pltpu.CompilerParams(
            dimension_semantics=("parallel",)),
    )(weight, img, txt, iid, tid, wi, wt, wii, wti)


_COMB_BLK = 1000
_ITEM_BLK0 = _USER // _COMB_BLK


def _combine_body(eh_ref, c1_ref, c2_ref, fp_ref, ip_ref, out_ref):
    i = pl.program_id(0)
    s0 = eh_ref[0] + c1_ref[0] + c2_ref[0]
    s1 = eh_ref[1] + c1_ref[1] + c2_ref[1]

    @pl.when(i >= _ITEM_BLK0)
    def _():
        out_ref[:, :_HALF] = s0 + fp_ref[0] + ip_ref[0]
        out_ref[:, _HALF:] = s1 + fp_ref[1] + ip_ref[1]

    @pl.when(i < _ITEM_BLK0)
    def _():
        out_ref[:, :_HALF] = s0
        out_ref[:, _HALF:] = s1


def _combine(embeds_h, cur1, cur2, featp, idp):
    n = _USER + _ITEM
    nblk = n // _COMB_BLK

    def item_map(i):
        return (0, lax.max(i - _ITEM_BLK0, 0), 0)

    return pl.pallas_call(
        _combine_body,
        grid=(nblk,),
        in_specs=[
            pl.BlockSpec((_NC, _COMB_BLK, _HALF), lambda i: (0, i, 0)),
            pl.BlockSpec((_NC, _COMB_BLK, _HALF), lambda i: (0, i, 0)),
            pl.BlockSpec((_NC, _COMB_BLK, _HALF), lambda i: (0, i, 0)),
            pl.BlockSpec((_NC, _COMB_BLK, _HALF), item_map),
            pl.BlockSpec((_NC, _COMB_BLK, _HALF), item_map),
        ],
        out_specs=pl.BlockSpec((_COMB_BLK, _LAT), lambda i: (i, 0)),
        out_shape=jax.ShapeDtypeStruct((n, _LAT), jnp.float32),
        compiler_params=pltpu.CompilerParams(
            dimension_semantics=("arbitrary",)),
    )(embeds_h, cur1, cur2, featp, idp)


def kernel(adj_index, adj_vals, id_adj_index, id_adj_vals, feature_adj_index,
           feature_adj_vals, image_embedding, text_embedding, image_id,
           text_id, uEmbeds, iEmbeds, image_trans, text_trans, image_id_trans,
           text_id_trans, modal_weight):
    weight = jax.nn.softmax(modal_weight, axis=0)

    feats_h, ids_h = _modal_transforms(
        weight, image_embedding, text_embedding, image_id, text_id,
        image_trans, text_trans, image_id_trans, text_id_trans)

    n = _USER + _ITEM
    n_pad = -(-n // 128) * 128
    item_pad = -(-_ITEM // 128) * 128
    embeds = jnp.concatenate(
        [uEmbeds, iEmbeds,
         jnp.zeros((n_pad - n, _LAT), jnp.float32)], axis=0)
    embeds_h = jnp.stack([embeds[:, :_HALF], embeds[:, _HALF:]])

    zeros_n = jnp.zeros((n_pad, _HALF), jnp.float32)
    zeros_i = jnp.zeros((item_pad, _HALF), jnp.float32)

    src, dst, vals, e_pad = _pad_edges(adj_index, adj_vals, n)
    spmm_main = _make_sc_spmm(n_pad, n_pad, e_pad)
    cur1 = spmm_main(embeds_h, src, dst, vals, zeros_n)
    cur2 = spmm_main(cur1, src, dst, vals, zeros_n)

    fsrc, fdst, fvals, fe_pad = _pad_edges(feature_adj_index,
                                           feature_adj_vals, _ITEM)
    isrc, idst, ivals, ie_pad = _pad_edges(id_adj_index, id_adj_vals, _ITEM)
    spmm_item = _make_sc_spmm(_ITEM, item_pad, fe_pad)
    featp = spmm_item(feats_h, fsrc, fdst, fvals, zeros_i)
    idp = spmm_item(ids_h, isrc, idst, ivals, zeros_i)

    return _combine(embeds_h, cur1, cur2, featp, idp)


# speedup vs baseline: 3.9065x; 1.2026x over previous
import dataclasses
import functools

import jax
import jax.numpy as jnp
from jax import lax
from jax.experimental import pallas as pl
from jax.experimental.pallas import tpu as pltpu
from jax.experimental.pallas import tpu_sc as plsc

_USER = 25000
_ITEM = 25000
_LAT = 64
_HALF = 32
_NC = 2
_NS = 16
_NW = _NC * _NS
_CHUNK = 128
_WIN = 16
_LANES = 16


@functools.lru_cache(maxsize=None)
def _make_sc_spmm(n_srcs, n_dst, e_pads):
    nt = len(n_srcs)
    chunks_ps = [e // (_NS * _CHUNK) for e in e_pads]
    rows_pt = n_dst // _NS
    mesh = plsc.VectorSubcoreMesh(core_axis_name="c", subcore_axis_name="s")

    def body(*refs):
        xs = refs[0:nt]
        srcs = refs[nt:2 * nt]
        dsts = refs[2 * nt:3 * nt]
        vals = refs[3 * nt:4 * nt]
        zero_hbm = refs[4 * nt]
        out_hbm = refs[4 * nt + 1]
        src_v, dst_v, val_v, rows0, rows1, acc_sh, sem0, sem1 = refs[4 * nt + 2:]
        rows_b = (rows0, rows1)
        sem_b = (sem0, sem1)

        c = lax.axis_index("c")
        s = lax.axis_index("s")

        pltpu.sync_copy(zero_hbm.at[pl.ds(s * rows_pt, rows_pt)],
                        acc_sh.at[pl.ds(s * rows_pt, rows_pt)])
        plsc.subcore_barrier()

        for t in range(nt):
            x_hbm, src_hbm, dst_hbm, val_hbm = xs[t], srcs[t], dsts[t], vals[t]
            cbase = s * chunks_ps[t]
            nwin = chunks_ps[t] // _WIN

            def gather(j, b, _x=x_hbm):
                pltpu.async_copy(_x.at[c].at[src_v.at[j]], rows_b[b],
                                 sem_b[b])

            def process(j, b):
                pltpu.make_async_copy(
                    x_hbm.at[c].at[src_v.at[j]], rows_b[b], sem_b[b]).wait()

                @pl.loop(0, _CHUNK, unroll=8)
                def _(e):
                    v = plsc.load_gather(
                        val_v, [jnp.full((_LANES,), j, jnp.int32),
                                jnp.full((_LANES,), e, jnp.int32)])
                    rows_b[b][e, pl.ds(0, _LANES)] = (
                        rows_b[b][e, pl.ds(0, _LANES)] * v)
                    rows_b[b][e, pl.ds(_LANES, _LANES)] = (
                        rows_b[b][e, pl.ds(_LANES, _LANES)] * v)

                pltpu.sync_copy(rows_b[b], acc_sh.at[dst_v.at[j]], add=True)

            @pl.loop(0, nwin)
            def _(w):
                wbase = cbase + w * _WIN
                pltpu.sync_copy(src_hbm.at[pl.ds(wbase, _WIN)], src_v)
                pltpu.sync_copy(dst_hbm.at[pl.ds(wbase, _WIN)], dst_v)
                pltpu.sync_copy(val_hbm.at[pl.ds(wbase, _WIN)], val_v)

                gather(0, 0)
                gather(1, 1)

                @pl.loop(0, _WIN, step=2)
                def _(j):
                    process(j, 0)

                    @pl.when(j + 2 < _WIN)
                    def _():
                        gather(j + 2, 0)

                    process(j + 1, 1)

                    @pl.when(j + 3 < _WIN)
                    def _():
                        gather(j + 3, 1)

        plsc.subcore_barrier()
        pltpu.sync_copy(acc_sh.at[pl.ds(s * rows_pt, rows_pt)],
                        out_hbm.at[c].at[pl.ds(s * rows_pt, rows_pt)])

    cp = pltpu.CompilerParams()
    for field, val in (("needs_layout_passes", False),
                       ("use_tc_tiling_on_sc", False)):
        if field in pltpu.CompilerParams.__dataclass_fields__:
            cp = dataclasses.replace(cp, **{field: val})
    kern = pl.kernel(
        body,
        out_type=jax.ShapeDtypeStruct((_NC, n_dst, _HALF), jnp.float32),
        mesh=mesh,
        compiler_params=cp,
        scratch_types=[
            pltpu.VMEM((_WIN, _CHUNK), jnp.int32),
            pltpu.VMEM((_WIN, _CHUNK), jnp.int32),
            pltpu.VMEM((_WIN, _CHUNK), jnp.float32),
            pltpu.VMEM((_CHUNK, _HALF), jnp.float32),
            pltpu.VMEM((_CHUNK, _HALF), jnp.float32),
            pltpu.VMEM_SHARED((n_dst, _HALF), jnp.float32),
            pltpu.SemaphoreType.DMA,
            pltpu.SemaphoreType.DMA,
        ],
    )

    def run(tables, edge_lists, zero):
        args = (list(tables)
                + [e[0] for e in edge_lists]
                + [e[1] for e in edge_lists]
                + [e[2] for e in edge_lists]
                + [zero])
        return kern(*args)

    return run


def _pad_edges(idx, vals, n_dst):
    e = idx.shape[1]
    group = _NS * _CHUNK * _WIN
    e_pad = -(-e // group) * group
    pad = e_pad - e
    dst = jnp.concatenate(
        [idx[0].astype(jnp.int32),
         jnp.arange(pad, dtype=jnp.int32) % n_dst])
    src = jnp.concatenate(
        [idx[1].astype(jnp.int32), jnp.zeros((pad,), jnp.int32)])
    v = jnp.concatenate([vals, jnp.zeros((pad,), jnp.float32)])
    return (src.reshape(-1, _CHUNK), dst.reshape(-1, _CHUNK),
            v.reshape(-1, _CHUNK), e_pad)


_FEAT_BLK = 200


def _feats_body(w_ref, img_ref, txt_ref, iid_ref, tid_ref,
                wi_ref, wt_ref, wii_ref, wti_ref, feats_ref, ids_ref):
    w0 = w_ref[0]
    w1 = w_ref[1]

    def leaky(y):
        return jnp.where(y > 0, y, 0.2 * y)

    fi = leaky(jnp.dot(img_ref[...].astype(jnp.bfloat16),
                       wi_ref[...].astype(jnp.bfloat16),
                       preferred_element_type=jnp.float32))
    ft = leaky(jnp.dot(txt_ref[...].astype(jnp.bfloat16),
                       wt_ref[...].astype(jnp.bfloat16),
                       preferred_element_type=jnp.float32))
    af = w0 * fi + w1 * ft
    feats_ref[0] = af[:, :_HALF]
    feats_ref[1] = af[:, _HALF:]

    ii = leaky(jnp.dot(iid_ref[...].astype(jnp.bfloat16),
                       wii_ref[...].astype(jnp.bfloat16),
                       preferred_element_type=jnp.float32))
    ti = leaky(jnp.dot(tid_ref[...].astype(jnp.bfloat16),
                       wti_ref[...].astype(jnp.bfloat16),
                       preferred_element_type=jnp.float32))
    ai = w0 * ii + w1 * ti
    ids_ref[0] = ai[:, :_HALF]
    ids_ref[1] = ai[:, _HALF:]


def _modal_transforms(weight, img, txt, iid, tid, wi, wt, wii, wti):
    nblk = _ITEM // _FEAT_BLK
    return pl.pallas_call(
        _feats_body,
        grid=(nblk,),
        in_specs=[
            pl.BlockSpec(memory_space=pltpu.SMEM),
            pl.BlockSpec((_FEAT_BLK, img.shape[1]), lambda i: (i, 0)),
            pl.BlockSpec((_FEAT_BLK, txt.shape[1]), lambda i: (i, 0)),
            pl.BlockSpec((_FEAT_BLK, _LAT), lambda i: (i, 0)),
            pl.BlockSpec((_FEAT_BLK, _LAT), lambda i: (i, 0)),
            pl.BlockSpec((img.shape[1], _LAT), lambda i: (0, 0)),
            pl.BlockSpec((txt.shape[1], _LAT), lambda i: (0, 0)),
            pl.BlockSpec((_LAT, _LAT), lambda i: (0, 0)),
            pl.BlockSpec((_LAT, _LAT), lambda i: (0, 0)),
        ],
        out_specs=[
            pl.BlockSpec((_NC, _FEAT_BLK, _HALF), lambda i: (0, i, 0)),
            pl.BlockSpec((_NC, _FEAT_BLK, _HALF), lambda i: (0, i, 0)),
        ],
        out_shape=[
            jax.ShapeDtypeStruct((_NC, _ITEM, _HALF), jnp.float32),
            jax.ShapeDtypeStruct((_NC, _ITEM, _HALF), jnp.float32),
        ],
        compiler_params=pltpu.CompilerParams(
            dimension_semantics=("parallel",)),
    )(weight, img, txt, iid, tid, wi, wt, wii, wti)


_COMB_BLK = 1000
_ITEM_BLK0 = _USER // _COMB_BLK


def _combine_body(eh_ref, c1_ref, c2_ref, fp_ref, out_ref):
    i = pl.program_id(0)
    s0 = eh_ref[0] + c1_ref[0] + c2_ref[0]
    s1 = eh_ref[1] + c1_ref[1] + c2_ref[1]

    @pl.when(i >= _ITEM_BLK0)
    def _():
        out_ref[:, :_HALF] = s0 + fp_ref[0]
        out_ref[:, _HALF:] = s1 + fp_ref[1]

    @pl.when(i < _ITEM_BLK0)
    def _():
        out_ref[:, :_HALF] = s0
        out_ref[:, _HALF:] = s1


def _combine(embeds_h, cur1, cur2, featp):
    n = _USER + _ITEM
    nblk = n // _COMB_BLK

    def item_map(i):
        return (0, lax.max(i - _ITEM_BLK0, 0), 0)

    return pl.pallas_call(
        _combine_body,
        grid=(nblk,),
        in_specs=[
            pl.BlockSpec((_NC, _COMB_BLK, _HALF), lambda i: (0, i, 0)),
            pl.BlockSpec((_NC, _COMB_BLK, _HALF), lambda i: (0, i, 0)),
            pl.BlockSpec((_NC, _COMB_BLK, _HALF), lambda i: (0, i, 0)),
            pl.BlockSpec((_NC, _COMB_BLK, _HALF), item_map),
        ],
        out_specs=pl.BlockSpec((_COMB_BLK, _LAT), lambda i: (i, 0)),
        out_shape=jax.ShapeDtypeStruct((n, _LAT), jnp.float32),
        compiler_params=pltpu.CompilerParams(
            dimension_semantics=("arbitrary",)),
    )(embeds_h, cur1, cur2, featp)


def kernel(adj_index, adj_vals, id_adj_index, id_adj_vals, feature_adj_index,
           feature_adj_vals, image_embedding, text_embedding, image_id,
           text_id, uEmbeds, iEmbeds, image_trans, text_trans, image_id_trans,
           text_id_trans, modal_weight):
    weight = jax.nn.softmax(modal_weight, axis=0)

    feats_h, ids_h = _modal_transforms(
        weight, image_embedding, text_embedding, image_id, text_id,
        image_trans, text_trans, image_id_trans, text_id_trans)

    n = _USER + _ITEM
    n_pad = -(-n // 128) * 128
    item_pad = -(-_ITEM // 128) * 128
    embeds = jnp.concatenate(
        [uEmbeds, iEmbeds,
         jnp.zeros((n_pad - n, _LAT), jnp.float32)], axis=0)
    embeds_h = jnp.stack([embeds[:, :_HALF], embeds[:, _HALF:]])

    zeros_n = jnp.zeros((n_pad, _HALF), jnp.float32)
    zeros_i = jnp.zeros((item_pad, _HALF), jnp.float32)

    src, dst, vals, e_pad = _pad_edges(adj_index, adj_vals, n)
    spmm_main = _make_sc_spmm((n_pad,), n_pad, (e_pad,))
    cur1 = spmm_main([embeds_h], [(src, dst, vals)], zeros_n)
    cur2 = spmm_main([cur1], [(src, dst, vals)], zeros_n)

    fsrc, fdst, fvals, fe_pad = _pad_edges(feature_adj_index,
                                           feature_adj_vals, _ITEM)
    isrc, idst, ivals, ie_pad = _pad_edges(id_adj_index, id_adj_vals, _ITEM)
    spmm_item = _make_sc_spmm((_ITEM, _ITEM), item_pad, (fe_pad, ie_pad))
    itemp = spmm_item([feats_h, ids_h],
                      [(fsrc, fdst, fvals), (isrc, idst, ivals)], zeros_i)

    return _combine(embeds_h, cur1, cur2, itemp)


# speedup vs baseline: 4.0131x; 1.0273x over previous
import dataclasses
import functools

import jax
import jax.numpy as jnp
from jax import lax
from jax.experimental import pallas as pl
from jax.experimental.pallas import tpu as pltpu
from jax.experimental.pallas import tpu_sc as plsc

_USER = 25000
_ITEM = 25000
_LAT = 64
_HALF = 32
_NC = 2
_NS = 16
_NW = _NC * _NS
_CHUNK = 128
_WIN = 16
_LANES = 16


@functools.lru_cache(maxsize=None)
def _make_sc_spmm(n_srcs, n_dst, e_pads):
    nt = len(n_srcs)
    chunks_ps = [e // (_NS * _CHUNK) for e in e_pads]
    rows_pt = n_dst // _NS
    mesh = plsc.VectorSubcoreMesh(core_axis_name="c", subcore_axis_name="s")

    def body(*refs):
        xs = refs[0:nt]
        srcs = refs[nt:2 * nt]
        dsts = refs[2 * nt:3 * nt]
        vals = refs[3 * nt:4 * nt]
        zero_hbm = refs[4 * nt]
        out_hbm = refs[4 * nt + 1]
        scratch = refs[4 * nt + 2:]
        src_v, dst_v, val_v = scratch[0:3]
        rows_b = scratch[3:7]
        acc_sh = scratch[7]
        gsem = scratch[8:12]
        ssem = scratch[12:16]

        c = lax.axis_index("c")
        s = lax.axis_index("s")

        pltpu.sync_copy(zero_hbm.at[pl.ds(s * rows_pt, rows_pt)],
                        acc_sh.at[pl.ds(s * rows_pt, rows_pt)])
        plsc.subcore_barrier()

        for t in range(nt):
            x_hbm, src_hbm, dst_hbm, val_hbm = xs[t], srcs[t], dsts[t], vals[t]
            cbase = s * chunks_ps[t]
            nwin = chunks_ps[t] // _WIN

            def gather(j, b, _x=x_hbm):
                pltpu.async_copy(_x.at[c].at[src_v.at[j]], rows_b[b], gsem[b])

            def wait_gather(j, b, _x=x_hbm):
                pltpu.make_async_copy(
                    _x.at[c].at[src_v.at[j]], rows_b[b], gsem[b]).wait()

            def scatter(j, b):
                pltpu.async_copy(rows_b[b], acc_sh.at[dst_v.at[j]], ssem[b],
                                 add=True)

            def wait_scatter(j, b):
                pltpu.make_async_copy(
                    rows_b[b], acc_sh.at[dst_v.at[j]], ssem[b]).wait()

            def multiply(j, b):
                @pl.loop(0, _CHUNK, unroll=8)
                def _(e):
                    v = plsc.load_gather(
                        val_v, [jnp.full((_LANES,), j, jnp.int32),
                                jnp.full((_LANES,), e, jnp.int32)])
                    rows_b[b][e, pl.ds(0, _LANES)] = (
                        rows_b[b][e, pl.ds(0, _LANES)] * v)
                    rows_b[b][e, pl.ds(_LANES, _LANES)] = (
                        rows_b[b][e, pl.ds(_LANES, _LANES)] * v)

            @pl.loop(0, nwin)
            def _(w):
                wbase = cbase + w * _WIN
                pltpu.sync_copy(src_hbm.at[pl.ds(wbase, _WIN)], src_v)
                pltpu.sync_copy(dst_hbm.at[pl.ds(wbase, _WIN)], dst_v)
                pltpu.sync_copy(val_hbm.at[pl.ds(wbase, _WIN)], val_v)

                gather(0, 0)
                gather(1, 1)

                @pl.loop(0, _WIN, step=4)
                def _(j):
                    for k in range(4):
                        i = j + k
                        b = k
                        bn = (k + 2) % 4
                        wait_gather(i, b)
                        multiply(i, b)
                        scatter(i, b)

                        @pl.when(i >= 2)
                        def _(i=i, bn=bn):
                            wait_scatter(i - 2, bn)

                        @pl.when(i + 2 < _WIN)
                        def _(i=i, bn=bn):
                            gather(i + 2, bn)

                wait_scatter(_WIN - 2, (_WIN - 2) % 4)
                wait_scatter(_WIN - 1, (_WIN - 1) % 4)

        plsc.subcore_barrier()
        pltpu.sync_copy(acc_sh.at[pl.ds(s * rows_pt, rows_pt)],
                        out_hbm.at[c].at[pl.ds(s * rows_pt, rows_pt)])

    cp = pltpu.CompilerParams()
    for field, val in (("needs_layout_passes", False),
                       ("use_tc_tiling_on_sc", False)):
        if field in pltpu.CompilerParams.__dataclass_fields__:
            cp = dataclasses.replace(cp, **{field: val})
    kern = pl.kernel(
        body,
        out_type=jax.ShapeDtypeStruct((_NC, n_dst, _HALF), jnp.float32),
        mesh=mesh,
        compiler_params=cp,
        scratch_types=(
            [pltpu.VMEM((_WIN, _CHUNK), jnp.int32),
             pltpu.VMEM((_WIN, _CHUNK), jnp.int32),
             pltpu.VMEM((_WIN, _CHUNK), jnp.float32)]
            + [pltpu.VMEM((_CHUNK, _HALF), jnp.float32)] * 4
            + [pltpu.VMEM_SHARED((n_dst, _HALF), jnp.float32)]
            + [pltpu.SemaphoreType.DMA] * 8
        ),
    )

    def run(tables, edge_lists, zero):
        args = (list(tables)
                + [e[0] for e in edge_lists]
                + [e[1] for e in edge_lists]
                + [e[2] for e in edge_lists]
                + [zero])
        return kern(*args)

    return run


def _pad_edges(idx, vals, n_dst):
    e = idx.shape[1]
    group = _NS * _CHUNK * _WIN
    e_pad = -(-e // group) * group
    pad = e_pad - e
    dst = jnp.concatenate(
        [idx[0].astype(jnp.int32),
         jnp.arange(pad, dtype=jnp.int32) % n_dst])
    src = jnp.concatenate(
        [idx[1].astype(jnp.int32), jnp.zeros((pad,), jnp.int32)])
    v = jnp.concatenate([vals, jnp.zeros((pad,), jnp.float32)])
    return (src.reshape(-1, _CHUNK), dst.reshape(-1, _CHUNK),
            v.reshape(-1, _CHUNK), e_pad)


_FEAT_BLK = 200


def _feats_body(w_ref, img_ref, txt_ref, iid_ref, tid_ref,
                wi_ref, wt_ref, wii_ref, wti_ref, feats_ref, ids_ref):
    w0 = w_ref[0]
    w1 = w_ref[1]

    def leaky(y):
        return jnp.where(y > 0, y, 0.2 * y)

    fi = leaky(jnp.dot(img_ref[...].astype(jnp.bfloat16),
                       wi_ref[...].astype(jnp.bfloat16),
                       preferred_element_type=jnp.float32))
    ft = leaky(jnp.dot(txt_ref[...].astype(jnp.bfloat16),
                       wt_ref[...].astype(jnp.bfloat16),
                       preferred_element_type=jnp.float32))
    af = w0 * fi + w1 * ft
    feats_ref[0] = af[:, :_HALF]
    feats_ref[1] = af[:, _HALF:]

    ii = leaky(jnp.dot(iid_ref[...].astype(jnp.bfloat16),
                       wii_ref[...].astype(jnp.bfloat16),
                       preferred_element_type=jnp.float32))
    ti = leaky(jnp.dot(tid_ref[...].astype(jnp.bfloat16),
                       wti_ref[...].astype(jnp.bfloat16),
                       preferred_element_type=jnp.float32))
    ai = w0 * ii + w1 * ti
    ids_ref[0] = ai[:, :_HALF]
    ids_ref[1] = ai[:, _HALF:]


def _modal_transforms(weight, img, txt, iid, tid, wi, wt, wii, wti):
    nblk = _ITEM // _FEAT_BLK
    return pl.pallas_call(
        _feats_body,
        grid=(nblk,),
        in_specs=[
            pl.BlockSpec(memory_space=pltpu.SMEM),
            pl.BlockSpec((_FEAT_BLK, img.shape[1]), lambda i: (i, 0)),
            pl.BlockSpec((_FEAT_BLK, txt.shape[1]), lambda i: (i, 0)),
            pl.BlockSpec((_FEAT_BLK, _LAT), lambda i: (i, 0)),
            pl.BlockSpec((_FEAT_BLK, _LAT), lambda i: (i, 0)),
            pl.BlockSpec((img.shape[1], _LAT), lambda i: (0, 0)),
            pl.BlockSpec((txt.shape[1], _LAT), lambda i: (0, 0)),
            pl.BlockSpec((_LAT, _LAT), lambda i: (0, 0)),
            pl.BlockSpec((_LAT, _LAT), lambda i: (0, 0)),
        ],
        out_specs=[
            pl.BlockSpec((_NC, _FEAT_BLK, _HALF), lambda i: (0, i, 0)),
            pl.BlockSpec((_NC, _FEAT_BLK, _HALF), lambda i: (0, i, 0)),
        ],
        out_shape=[
            jax.ShapeDtypeStruct((_NC, _ITEM, _HALF), jnp.float32),
            jax.ShapeDtypeStruct((_NC, _ITEM, _HALF), jnp.float32),
        ],
        compiler_params=pltpu.CompilerParams(
            dimension_semantics=("parallel",)),
    )(weight, img, txt, iid, tid, wi, wt, wii, wti)


_COMB_BLK = 1000
_ITEM_BLK0 = _USER // _COMB_BLK


def _combine_body(eh_ref, c1_ref, c2_ref, fp_ref, out_ref):
    i = pl.program_id(0)
    s0 = eh_ref[0] + c1_ref[0] + c2_ref[0]
    s1 = eh_ref[1] + c1_ref[1] + c2_ref[1]

    @pl.when(i >= _ITEM_BLK0)
    def _():
        out_ref[:, :_HALF] = s0 + fp_ref[0]
        out_ref[:, _HALF:] = s1 + fp_ref[1]

    @pl.when(i < _ITEM_BLK0)
    def _():
        out_ref[:, :_HALF] = s0
        out_ref[:, _HALF:] = s1


def _combine(embeds_h, cur1, cur2, featp):
    n = _USER + _ITEM
    nblk = n // _COMB_BLK

    def item_map(i):
        return (0, lax.max(i - _ITEM_BLK0, 0), 0)

    return pl.pallas_call(
        _combine_body,
        grid=(nblk,),
        in_specs=[
            pl.BlockSpec((_NC, _COMB_BLK, _HALF), lambda i: (0, i, 0)),
            pl.BlockSpec((_NC, _COMB_BLK, _HALF), lambda i: (0, i, 0)),
            pl.BlockSpec((_NC, _COMB_BLK, _HALF), lambda i: (0, i, 0)),
            pl.BlockSpec((_NC, _COMB_BLK, _HALF), item_map),
        ],
        out_specs=pl.BlockSpec((_COMB_BLK, _LAT), lambda i: (i, 0)),
        out_shape=jax.ShapeDtypeStruct((n, _LAT), jnp.float32),
        compiler_params=pltpu.CompilerParams(
            dimension_semantics=("arbitrary",)),
    )(embeds_h, cur1, cur2, featp)


def kernel(adj_index, adj_vals, id_adj_index, id_adj_vals, feature_adj_index,
           feature_adj_vals, image_embedding, text_embedding, image_id,
           text_id, uEmbeds, iEmbeds, image_trans, text_trans, image_id_trans,
           text_id_trans, modal_weight):
    weight = jax.nn.softmax(modal_weight, axis=0)

    feats_h, ids_h = _modal_transforms(
        weight, image_embedding, text_embedding, image_id, text_id,
        image_trans, text_trans, image_id_trans, text_id_trans)

    n = _USER + _ITEM
    n_pad = -(-n // 128) * 128
    item_pad = -(-_ITEM // 128) * 128
    embeds = jnp.concatenate(
        [uEmbeds, iEmbeds,
         jnp.zeros((n_pad - n, _LAT), jnp.float32)], axis=0)
    embeds_h = jnp.stack([embeds[:, :_HALF], embeds[:, _HALF:]])

    zeros_n = jnp.zeros((n_pad, _HALF), jnp.float32)
    zeros_i = jnp.zeros((item_pad, _HALF), jnp.float32)

    src, dst, vals, e_pad = _pad_edges(adj_index, adj_vals, n)
    spmm_main = _make_sc_spmm((n_pad,), n_pad, (e_pad,))
    cur1 = spmm_main([embeds_h], [(src, dst, vals)], zeros_n)
    cur2 = spmm_main([cur1], [(src, dst, vals)], zeros_n)

    fsrc, fdst, fvals, fe_pad = _pad_edges(feature_adj_index,
                                           feature_adj_vals, _ITEM)
    isrc, idst, ivals, ie_pad = _pad_edges(id_adj_index, id_adj_vals, _ITEM)
    spmm_item = _make_sc_spmm((_ITEM, _ITEM), item_pad, (fe_pad, ie_pad))
    itemp = spmm_item([feats_h, ids_h],
                      [(fsrc, fdst, fvals), (isrc, idst, ivals)], zeros_i)

    return _combine(embeds_h, cur1, cur2, itemp)


# speedup vs baseline: 4.2720x; 1.0645x over previous
import dataclasses
import functools

import jax
import jax.numpy as jnp
from jax import lax
from jax.experimental import pallas as pl
from jax.experimental.pallas import tpu as pltpu
from jax.experimental.pallas import tpu_sc as plsc

_USER = 25000
_ITEM = 25000
_LAT = 64
_HALF = 32
_NC = 2
_NS = 16
_NW = _NC * _NS
_CHUNK = 128
_WIN = 16
_LANES = 16


@functools.lru_cache(maxsize=None)
def _make_sc_spmm(n_srcs, n_dst, e_pads):
    nt = len(n_srcs)
    chunks_ps = [e // (_NS * _CHUNK) for e in e_pads]
    rows_pt = n_dst // _NS
    mesh = plsc.VectorSubcoreMesh(core_axis_name="c", subcore_axis_name="s")

    def body(*refs):
        xs = refs[0:nt]
        srcs = refs[nt:2 * nt]
        dsts = refs[2 * nt:3 * nt]
        vals = refs[3 * nt:4 * nt]
        zero_hbm = refs[4 * nt]
        out_hbm = refs[4 * nt + 1]
        scratch = refs[4 * nt + 2:]
        src_v, dst_v, val_v = scratch[0:3]
        rows_b = scratch[3:7]
        acc_sh = scratch[7]
        gsem = scratch[8:12]
        ssem = scratch[12:16]

        c = lax.axis_index("c")
        s = lax.axis_index("s")

        pltpu.sync_copy(zero_hbm.at[pl.ds(s * rows_pt, rows_pt)],
                        acc_sh.at[pl.ds(s * rows_pt, rows_pt)])
        plsc.subcore_barrier()

        for t in range(nt):
            x_hbm, src_hbm, dst_hbm, val_hbm = xs[t], srcs[t], dsts[t], vals[t]
            cbase = s * chunks_ps[t]
            nwin = chunks_ps[t] // _WIN

            def gather(j, b, _x=x_hbm):
                pltpu.async_copy(_x.at[c].at[src_v.at[j]], rows_b[b], gsem[b])

            def wait_gather(j, b, _x=x_hbm):
                pltpu.make_async_copy(
                    _x.at[c].at[src_v.at[j]], rows_b[b], gsem[b]).wait()

            def scatter(j, b):
                pltpu.async_copy(rows_b[b], acc_sh.at[dst_v.at[j]], ssem[b],
                                 add=True)

            def wait_scatter(j, b):
                pltpu.make_async_copy(
                    rows_b[b], acc_sh.at[dst_v.at[j]], ssem[b]).wait()

            def multiply(j, b):
                rb = rows_b[b]

                dnums = lax.GatherDimensionNumbers(
                    offset_dims=(), collapsed_slice_dims=(0,),
                    start_index_map=(0,))

                @pl.loop(0, _CHUNK, step=_LANES)
                def _(e0):
                    vv = val_v[j, pl.ds(e0, _LANES)]
                    for e16 in range(_LANES):
                        v = lax.gather(
                            vv,
                            jnp.full((_LANES, 1), e16, jnp.int32),
                            dimension_numbers=dnums, slice_sizes=(1,),
                            mode=lax.GatherScatterMode.PROMISE_IN_BOUNDS)
                        e = e0 + e16
                        rb[e, pl.ds(0, _LANES)] = (
                            rb[e, pl.ds(0, _LANES)] * v)
                        rb[e, pl.ds(_LANES, _LANES)] = (
                            rb[e, pl.ds(_LANES, _LANES)] * v)

            @pl.loop(0, nwin)
            def _(w):
                wbase = cbase + w * _WIN
                pltpu.sync_copy(src_hbm.at[pl.ds(wbase, _WIN)], src_v)
                pltpu.sync_copy(dst_hbm.at[pl.ds(wbase, _WIN)], dst_v)
                pltpu.sync_copy(val_hbm.at[pl.ds(wbase, _WIN)], val_v)

                gather(0, 0)
                gather(1, 1)

                @pl.loop(0, _WIN, step=4)
                def _(j):
                    for k in range(4):
                        i = j + k
                        b = k
                        bn = (k + 2) % 4
                        wait_gather(i, b)
                        multiply(i, b)
                        scatter(i, b)

                        @pl.when(i >= 2)
                        def _(i=i, bn=bn):
                            wait_scatter(i - 2, bn)

                        @pl.when(i + 2 < _WIN)
                        def _(i=i, bn=bn):
                            gather(i + 2, bn)

                wait_scatter(_WIN - 2, (_WIN - 2) % 4)
                wait_scatter(_WIN - 1, (_WIN - 1) % 4)

        plsc.subcore_barrier()
        pltpu.sync_copy(acc_sh.at[pl.ds(s * rows_pt, rows_pt)],
                        out_hbm.at[c].at[pl.ds(s * rows_pt, rows_pt)])

    cp = pltpu.CompilerParams()
    for field, val in (("needs_layout_passes", False),
                       ("use_tc_tiling_on_sc", False)):
        if field in pltpu.CompilerParams.__dataclass_fields__:
            cp = dataclasses.replace(cp, **{field: val})
    kern = pl.kernel(
        body,
        out_type=jax.ShapeDtypeStruct((_NC, n_dst, _HALF), jnp.float32),
        mesh=mesh,
        compiler_params=cp,
        scratch_types=(
            [pltpu.VMEM((_WIN, _CHUNK), jnp.int32),
             pltpu.VMEM((_WIN, _CHUNK), jnp.int32),
             pltpu.VMEM((_WIN, _CHUNK), jnp.float32)]
            + [pltpu.VMEM((_CHUNK, _HALF), jnp.float32)] * 4
            + [pltpu.VMEM_SHARED((n_dst, _HALF), jnp.float32)]
            + [pltpu.SemaphoreType.DMA] * 8
        ),
    )

    def run(tables, edge_lists, zero):
        args = (list(tables)
                + [e[0] for e in edge_lists]
                + [e[1] for e in edge_lists]
                + [e[2] for e in edge_lists]
                + [zero])
        return kern(*args)

    return run


def _pad_edges(idx, vals, n_dst):
    e = idx.shape[1]
    group = _NS * _CHUNK * _WIN
    e_pad = -(-e // group) * group
    pad = e_pad - e
    dst = jnp.concatenate(
        [idx[0].astype(jnp.int32),
         jnp.arange(pad, dtype=jnp.int32) % n_dst])
    src = jnp.concatenate(
        [idx[1].astype(jnp.int32), jnp.zeros((pad,), jnp.int32)])
    v = jnp.concatenate([vals, jnp.zeros((pad,), jnp.float32)])
    return (src.reshape(-1, _CHUNK), dst.reshape(-1, _CHUNK),
            v.reshape(-1, _CHUNK), e_pad)


_FEAT_BLK = 200


def _feats_body(w_ref, img_ref, txt_ref, iid_ref, tid_ref,
                wi_ref, wt_ref, wii_ref, wti_ref, feats_ref, ids_ref):
    w0 = w_ref[0]
    w1 = w_ref[1]

    def leaky(y):
        return jnp.where(y > 0, y, 0.2 * y)

    fi = leaky(jnp.dot(img_ref[...].astype(jnp.bfloat16),
                       wi_ref[...].astype(jnp.bfloat16),
                       preferred_element_type=jnp.float32))
    ft = leaky(jnp.dot(txt_ref[...].astype(jnp.bfloat16),
                       wt_ref[...].astype(jnp.bfloat16),
                       preferred_element_type=jnp.float32))
    af = w0 * fi + w1 * ft
    feats_ref[0] = af[:, :_HALF]
    feats_ref[1] = af[:, _HALF:]

    ii = leaky(jnp.dot(iid_ref[...].astype(jnp.bfloat16),
                       wii_ref[...].astype(jnp.bfloat16),
                       preferred_element_type=jnp.float32))
    ti = leaky(jnp.dot(tid_ref[...].astype(jnp.bfloat16),
                       wti_ref[...].astype(jnp.bfloat16),
                       preferred_element_type=jnp.float32))
    ai = w0 * ii + w1 * ti
    ids_ref[0] = ai[:, :_HALF]
    ids_ref[1] = ai[:, _HALF:]


def _modal_transforms(weight, img, txt, iid, tid, wi, wt, wii, wti):
    nblk = _ITEM // _FEAT_BLK
    return pl.pallas_call(
        _feats_body,
        grid=(nblk,),
        in_specs=[
            pl.BlockSpec(memory_space=pltpu.SMEM),
            pl.BlockSpec((_FEAT_BLK, img.shape[1]), lambda i: (i, 0)),
            pl.BlockSpec((_FEAT_BLK, txt.shape[1]), lambda i: (i, 0)),
            pl.BlockSpec((_FEAT_BLK, _LAT), lambda i: (i, 0)),
            pl.BlockSpec((_FEAT_BLK, _LAT), lambda i: (i, 0)),
            pl.BlockSpec((img.shape[1], _LAT), lambda i: (0, 0)),
            pl.BlockSpec((txt.shape[1], _LAT), lambda i: (0, 0)),
            pl.BlockSpec((_LAT, _LAT), lambda i: (0, 0)),
            pl.BlockSpec((_LAT, _LAT), lambda i: (0, 0)),
        ],
        out_specs=[
            pl.BlockSpec((_NC, _FEAT_BLK, _HALF), lambda i: (0, i, 0)),
            pl.BlockSpec((_NC, _FEAT_BLK, _HALF), lambda i: (0, i, 0)),
        ],
        out_shape=[
            jax.ShapeDtypeStruct((_NC, _ITEM, _HALF), jnp.float32),
            jax.ShapeDtypeStruct((_NC, _ITEM, _HALF), jnp.float32),
        ],
        compiler_params=pltpu.CompilerParams(
            dimension_semantics=("parallel",)),
    )(weight, img, txt, iid, tid, wi, wt, wii, wti)


_COMB_BLK = 1000
_ITEM_BLK0 = _USER // _COMB_BLK


def _combine_body(eh_ref, c1_ref, c2_ref, fp_ref, out_ref):
    i = pl.program_id(0)
    s0 = eh_ref[0] + c1_ref[0] + c2_ref[0]
    s1 = eh_ref[1] + c1_ref[1] + c2_ref[1]

    @pl.when(i >= _ITEM_BLK0)
    def _():
        out_ref[:, :_HALF] = s0 + fp_ref[0]
        out_ref[:, _HALF:] = s1 + fp_ref[1]

    @pl.when(i < _ITEM_BLK0)
    def _():
        out_ref[:, :_HALF] = s0
        out_ref[:, _HALF:] = s1


def _combine(embeds_h, cur1, cur2, featp):
    n = _USER + _ITEM
    nblk = n // _COMB_BLK

    def item_map(i):
        return (0, lax.max(i - _ITEM_BLK0, 0), 0)

    return pl.pallas_call(
        _combine_body,
        grid=(nblk,),
        in_specs=[
            pl.BlockSpec((_NC, _COMB_BLK, _HALF), lambda i: (0, i, 0)),
            pl.BlockSpec((_NC, _COMB_BLK, _HALF), lambda i: (0, i, 0)),
            pl.BlockSpec((_NC, _COMB_BLK, _HALF), lambda i: (0, i, 0)),
            pl.BlockSpec((_NC, _COMB_BLK, _HALF), item_map),
        ],
        out_specs=pl.BlockSpec((_COMB_BLK, _LAT), lambda i: (i, 0)),
        out_shape=jax.ShapeDtypeStruct((n, _LAT), jnp.float32),
        compiler_params=pltpu.CompilerParams(
            dimension_semantics=("arbitrary",)),
    )(embeds_h, cur1, cur2, featp)


def kernel(adj_index, adj_vals, id_adj_index, id_adj_vals, feature_adj_index,
           feature_adj_vals, image_embedding, text_embedding, image_id,
           text_id, uEmbeds, iEmbeds, image_trans, text_trans, image_id_trans,
           text_id_trans, modal_weight):
    weight = jax.nn.softmax(modal_weight, axis=0)

    feats_h, ids_h = _modal_transforms(
        weight, image_embedding, text_embedding, image_id, text_id,
        image_trans, text_trans, image_id_trans, text_id_trans)

    n = _USER + _ITEM
    n_pad = -(-n // 128) * 128
    item_pad = -(-_ITEM // 128) * 128
    embeds = jnp.concatenate(
        [uEmbeds, iEmbeds,
         jnp.zeros((n_pad - n, _LAT), jnp.float32)], axis=0)
    embeds_h = jnp.stack([embeds[:, :_HALF], embeds[:, _HALF:]])

    zeros_n = jnp.zeros((n_pad, _HALF), jnp.float32)
    zeros_i = jnp.zeros((item_pad, _HALF), jnp.float32)

    src, dst, vals, e_pad = _pad_edges(adj_index, adj_vals, n)
    spmm_main = _make_sc_spmm((n_pad,), n_pad, (e_pad,))
    cur1 = spmm_main([embeds_h], [(src, dst, vals)], zeros_n)
    cur2 = spmm_main([cur1], [(src, dst, vals)], zeros_n)

    fsrc, fdst, fvals, fe_pad = _pad_edges(feature_adj_index,
                                           feature_adj_vals, _ITEM)
    isrc, idst, ivals, ie_pad = _pad_edges(id_adj_index, id_adj_vals, _ITEM)
    spmm_item = _make_sc_spmm((_ITEM, _ITEM), item_pad, (fe_pad, ie_pad))
    itemp = spmm_item([feats_h, ids_h],
                      [(fsrc, fdst, fvals), (isrc, idst, ivals)], zeros_i)

    return _combine(embeds_h, cur1, cur2, itemp)


# speedup vs baseline: 6.8549x; 1.6046x over previous
import dataclasses
import functools

import jax
import jax.numpy as jnp
from jax import lax
from jax.experimental import pallas as pl
from jax.experimental.pallas import tpu as pltpu
from jax.experimental.pallas import tpu_sc as plsc

_USER = 25000
_ITEM = 25000
_LAT = 64
_HALF = 32
_NC = 2
_NS = 16
_NW = _NC * _NS
_CHUNK = 128
_WIN = 16
_LANES = 16
_QCOL = 16


@functools.lru_cache(maxsize=None)
def _make_sc_spmm(n_srcs, n_dst, e_pads):
    nt = len(n_srcs)
    chunks_ps = [e // (_NS * _CHUNK) for e in e_pads]
    rows_pt = n_dst // _NS
    mesh = plsc.VectorSubcoreMesh(core_axis_name="c", subcore_axis_name="s")

    n_src = n_srcs[0]
    assert all(x == n_src for x in n_srcs)
    srows_pt = n_src // _NS

    def body(*refs):
        xs = refs[0:nt]
        srcs = refs[nt:2 * nt]
        dsts = refs[2 * nt:3 * nt]
        vals = refs[3 * nt:4 * nt]
        zero_hbm = refs[4 * nt]
        out_hbm = refs[4 * nt + 1]
        scratch = refs[4 * nt + 2:]
        src_v, dst_v, val_v = scratch[0:3]
        rows_b = scratch[3:7]
        tab_sh = scratch[7]
        acc_sh = scratch[8]
        gsem = scratch[9:13]
        ssem = scratch[13:17]

        c = lax.axis_index("c")
        s = lax.axis_index("s")

        dnums = lax.GatherDimensionNumbers(
            offset_dims=(), collapsed_slice_dims=(0,), start_index_map=(0,))

        def gather(j, b):
            pltpu.async_copy(tab_sh.at[src_v.at[j]], rows_b[b], gsem[b])

        def wait_gather(j, b):
            pltpu.make_async_copy(
                tab_sh.at[src_v.at[j]], rows_b[b], gsem[b]).wait()

        def scatter(j, b):
            pltpu.async_copy(rows_b[b], acc_sh.at[dst_v.at[j]], ssem[b],
                             add=True)

        def wait_scatter(j, b):
            pltpu.make_async_copy(
                rows_b[b], acc_sh.at[dst_v.at[j]], ssem[b]).wait()

        def multiply(j, b):
            rb = rows_b[b]

            @pl.loop(0, _CHUNK, step=_LANES)
            def _(e0):
                vv = val_v[j, pl.ds(e0, _LANES)]
                for e16 in range(_LANES):
                    v = lax.gather(
                        vv,
                        jnp.full((_LANES, 1), e16, jnp.int32),
                        dimension_numbers=dnums, slice_sizes=(1,),
                        mode=lax.GatherScatterMode.PROMISE_IN_BOUNDS)
                    rb[e0 + e16, pl.ds(0, _LANES)] = (
                        rb[e0 + e16, pl.ds(0, _LANES)] * v)

        for p in range(2):
            pltpu.sync_copy(zero_hbm.at[pl.ds(s * rows_pt, rows_pt)],
                            acc_sh.at[pl.ds(s * rows_pt, rows_pt)])
            pltpu.sync_copy(
                xs[0].at[2 * c + p].at[pl.ds(s * srows_pt, srows_pt)],
                tab_sh.at[pl.ds(s * srows_pt, srows_pt)])
            plsc.subcore_barrier()

            for t in range(nt):
                if t > 0:
                    pltpu.sync_copy(
                        xs[t].at[2 * c + p].at[pl.ds(s * srows_pt,
                                                     srows_pt)],
                        tab_sh.at[pl.ds(s * srows_pt, srows_pt)])
                    plsc.subcore_barrier()
                src_hbm, dst_hbm, val_hbm = srcs[t], dsts[t], vals[t]
                cbase = s * chunks_ps[t]
                nwin = chunks_ps[t] // _WIN

                @pl.loop(0, nwin)
                def _(w):
                    wbase = cbase + w * _WIN
                    pltpu.sync_copy(src_hbm.at[pl.ds(wbase, _WIN)], src_v)
                    pltpu.sync_copy(dst_hbm.at[pl.ds(wbase, _WIN)], dst_v)
                    pltpu.sync_copy(val_hbm.at[pl.ds(wbase, _WIN)], val_v)

                    gather(0, 0)
                    gather(1, 1)

                    @pl.loop(0, _WIN, step=4)
                    def _(j):
                        for k in range(4):
                            i = j + k
                            b = k
                            bn = (k + 2) % 4
                            wait_gather(i, b)
                            multiply(i, b)
                            scatter(i, b)

                            @pl.when(i >= 2)
                            def _(i=i, bn=bn):
                                wait_scatter(i - 2, bn)

                            @pl.when(i + 2 < _WIN)
                            def _(i=i, bn=bn):
                                gather(i + 2, bn)

                    wait_scatter(_WIN - 2, (_WIN - 2) % 4)
                    wait_scatter(_WIN - 1, (_WIN - 1) % 4)

                plsc.subcore_barrier()

            pltpu.sync_copy(acc_sh.at[pl.ds(s * rows_pt, rows_pt)],
                            out_hbm.at[2 * c + p].at[pl.ds(s * rows_pt,
                                                           rows_pt)])
            if p == 0:
                plsc.subcore_barrier()

    cp = pltpu.CompilerParams()
    for field, val in (("needs_layout_passes", False),
                       ("use_tc_tiling_on_sc", False)):
        if field in pltpu.CompilerParams.__dataclass_fields__:
            cp = dataclasses.replace(cp, **{field: val})
    kern = pl.kernel(
        body,
        out_type=jax.ShapeDtypeStruct((2 * _NC, n_dst, _QCOL), jnp.float32),
        mesh=mesh,
        compiler_params=cp,
        scratch_types=(
            [pltpu.VMEM((_WIN, _CHUNK), jnp.int32),
             pltpu.VMEM((_WIN, _CHUNK), jnp.int32),
             pltpu.VMEM((_WIN, _CHUNK), jnp.float32)]
            + [pltpu.VMEM((_CHUNK, _QCOL), jnp.float32)] * 4
            + [pltpu.VMEM_SHARED((n_src, _QCOL), jnp.float32),
               pltpu.VMEM_SHARED((n_dst, _QCOL), jnp.float32)]
            + [pltpu.SemaphoreType.DMA] * 8
        ),
    )

    def run(tables, edge_lists, zero):
        args = (list(tables)
                + [e[0] for e in edge_lists]
                + [e[1] for e in edge_lists]
                + [e[2] for e in edge_lists]
                + [zero])
        return kern(*args)

    return run


def _pad_edges(idx, vals, n_dst):
    e = idx.shape[1]
    group = _NS * _CHUNK * _WIN
    e_pad = -(-e // group) * group
    pad = e_pad - e
    dst = jnp.concatenate(
        [idx[0].astype(jnp.int32),
         jnp.arange(pad, dtype=jnp.int32) % n_dst])
    src = jnp.concatenate(
        [idx[1].astype(jnp.int32), jnp.zeros((pad,), jnp.int32)])
    v = jnp.concatenate([vals, jnp.zeros((pad,), jnp.float32)])
    return (src.reshape(-1, _CHUNK), dst.reshape(-1, _CHUNK),
            v.reshape(-1, _CHUNK), e_pad)


_FEAT_BLK = 224


def _feats_body(w_ref, img_ref, txt_ref, iid_ref, tid_ref,
                wi_ref, wt_ref, wii_ref, wti_ref, feats_ref, ids_ref):
    w0 = w_ref[0]
    w1 = w_ref[1]

    def leaky(y):
        return jnp.where(y > 0, y, 0.2 * y)

    fi = leaky(jnp.dot(img_ref[...].astype(jnp.bfloat16),
                       wi_ref[...].astype(jnp.bfloat16),
                       preferred_element_type=jnp.float32))
    ft = leaky(jnp.dot(txt_ref[...].astype(jnp.bfloat16),
                       wt_ref[...].astype(jnp.bfloat16),
                       preferred_element_type=jnp.float32))
    af = w0 * fi + w1 * ft

    ii = leaky(jnp.dot(iid_ref[...].astype(jnp.bfloat16),
                       wii_ref[...].astype(jnp.bfloat16),
                       preferred_element_type=jnp.float32))
    ti = leaky(jnp.dot(tid_ref[...].astype(jnp.bfloat16),
                       wti_ref[...].astype(jnp.bfloat16),
                       preferred_element_type=jnp.float32))
    ai = w0 * ii + w1 * ti

    for q in range(4):
        feats_ref[q] = af[:, q * _QCOL:(q + 1) * _QCOL]
        ids_ref[q] = ai[:, q * _QCOL:(q + 1) * _QCOL]


def _modal_transforms(weight, img, txt, iid, tid, wi, wt, wii, wti):
    item_pad = -(-_ITEM // 128) * 128
    nblk = item_pad // _FEAT_BLK
    return pl.pallas_call(
        _feats_body,
        grid=(nblk,),
        in_specs=[
            pl.BlockSpec(memory_space=pltpu.SMEM),
            pl.BlockSpec((_FEAT_BLK, img.shape[1]), lambda i: (i, 0)),
            pl.BlockSpec((_FEAT_BLK, txt.shape[1]), lambda i: (i, 0)),
            pl.BlockSpec((_FEAT_BLK, _LAT), lambda i: (i, 0)),
            pl.BlockSpec((_FEAT_BLK, _LAT), lambda i: (i, 0)),
            pl.BlockSpec((img.shape[1], _LAT), lambda i: (0, 0)),
            pl.BlockSpec((txt.shape[1], _LAT), lambda i: (0, 0)),
            pl.BlockSpec((_LAT, _LAT), lambda i: (0, 0)),
            pl.BlockSpec((_LAT, _LAT), lambda i: (0, 0)),
        ],
        out_specs=[
            pl.BlockSpec((4, _FEAT_BLK, _QCOL), lambda i: (0, i, 0)),
            pl.BlockSpec((4, _FEAT_BLK, _QCOL), lambda i: (0, i, 0)),
        ],
        out_shape=[
            jax.ShapeDtypeStruct((4, item_pad, _QCOL), jnp.float32),
            jax.ShapeDtypeStruct((4, item_pad, _QCOL), jnp.float32),
        ],
        compiler_params=pltpu.CompilerParams(
            dimension_semantics=("parallel",)),
    )(weight, img, txt, iid, tid, wi, wt, wii, wti)


_COMB_BLK = 1000
_ITEM_BLK0 = _USER // _COMB_BLK


def _combine_body(eh_ref, c1_ref, c2_ref, fp_ref, out_ref):
    i = pl.program_id(0)
    for q in range(4):
        sq = eh_ref[q] + c1_ref[q] + c2_ref[q]
        cols = pl.ds(q * _QCOL, _QCOL)

        @pl.when(i >= _ITEM_BLK0)
        def _(sq=sq, cols=cols, q=q):
            out_ref[:, cols] = sq + fp_ref[q]

        @pl.when(i < _ITEM_BLK0)
        def _(sq=sq, cols=cols):
            out_ref[:, cols] = sq


def _combine(embeds_h, cur1, cur2, featp):
    n = _USER + _ITEM
    nblk = n // _COMB_BLK

    def item_map(i):
        return (0, lax.max(i - _ITEM_BLK0, 0), 0)

    return pl.pallas_call(
        _combine_body,
        grid=(nblk,),
        in_specs=[
            pl.BlockSpec((4, _COMB_BLK, _QCOL), lambda i: (0, i, 0)),
            pl.BlockSpec((4, _COMB_BLK, _QCOL), lambda i: (0, i, 0)),
            pl.BlockSpec((4, _COMB_BLK, _QCOL), lambda i: (0, i, 0)),
            pl.BlockSpec((4, _COMB_BLK, _QCOL), item_map),
        ],
        out_specs=pl.BlockSpec((_COMB_BLK, _LAT), lambda i: (i, 0)),
        out_shape=jax.ShapeDtypeStruct((n, _LAT), jnp.float32),
        compiler_params=pltpu.CompilerParams(
            dimension_semantics=("arbitrary",)),
    )(embeds_h, cur1, cur2, featp)


def kernel(adj_index, adj_vals, id_adj_index, id_adj_vals, feature_adj_index,
           feature_adj_vals, image_embedding, text_embedding, image_id,
           text_id, uEmbeds, iEmbeds, image_trans, text_trans, image_id_trans,
           text_id_trans, modal_weight):
    weight = jax.nn.softmax(modal_weight, axis=0)

    feats_h, ids_h = _modal_transforms(
        weight, image_embedding, text_embedding, image_id, text_id,
        image_trans, text_trans, image_id_trans, text_id_trans)

    n = _USER + _ITEM
    n_pad = -(-n // 128) * 128
    item_pad = -(-_ITEM // 128) * 128
    embeds = jnp.concatenate(
        [uEmbeds, iEmbeds,
         jnp.zeros((n_pad - n, _LAT), jnp.float32)], axis=0)
    embeds_h = jnp.stack(
        [embeds[:, q * _QCOL:(q + 1) * _QCOL] for q in range(4)])

    zeros_n = jnp.zeros((n_pad, _QCOL), jnp.float32)
    zeros_i = jnp.zeros((item_pad, _QCOL), jnp.float32)

    src, dst, vals, e_pad = _pad_edges(adj_index, adj_vals, n)
    spmm_main = _make_sc_spmm((n_pad,), n_pad, (e_pad,))
    cur1 = spmm_main([embeds_h], [(src, dst, vals)], zeros_n)
    cur2 = spmm_main([cur1], [(src, dst, vals)], zeros_n)

    fsrc, fdst, fvals, fe_pad = _pad_edges(feature_adj_index,
                                           feature_adj_vals, _ITEM)
    isrc, idst, ivals, ie_pad = _pad_edges(id_adj_index, id_adj_vals, _ITEM)
    spmm_item = _make_sc_spmm((item_pad, item_pad), item_pad,
                              (fe_pad, ie_pad))
    itemp = spmm_item([feats_h, ids_h],
                      [(fsrc, fdst, fvals), (isrc, idst, ivals)], zeros_i)

    return _combine(embeds_h, cur1, cur2, itemp)
